# baseline (speedup 1.0000x reference)
import jax
import jax.numpy as jnp
from jax.experimental import pallas as pl
from jax.experimental.pallas import tpu as pltpu

B, C, G = 1024, 8, 4096
D0, D1, D2 = 512, 2048, 256
K = 1024
COM_COST = 0.25


def _enc_body(x_ref, w_ref, b_ref, h_ref):
    x = x_ref[...]
    w = w_ref[0]
    acc = jnp.dot(x, w, preferred_element_type=jnp.float32)
    h_ref[...] = jnp.tanh(acc + b_ref[0])


def _z1_body(h_ref, w_ref, b_ref, z1_ref):
    acc = jnp.dot(h_ref[...], w_ref[...], preferred_element_type=jnp.float32)
    z1_ref[...] = jnp.tanh(acc + b_ref[...])


def _mid_body(z1_ref, w2_ref, b2_ref, cb_ref, dw1_ref, db1_ref,
              loss_ref, qst_ref, d1_ref):
    z = jnp.tanh(jnp.dot(z1_ref[...], w2_ref[...],
                         preferred_element_type=jnp.float32) + b2_ref[...])
    cb = cb_ref[...]
    zsq = jnp.sum(z * z, axis=1, keepdims=True)
    csq = jnp.sum(cb * cb, axis=1, keepdims=True)
    p = jax.lax.dot_general(z, cb, (((1,), (1,)), ((), ())),
                            preferred_element_type=jnp.float32)
    dist = (zsq + csq.T) - 2.0 * p
    rowmin = jnp.min(dist, axis=1, keepdims=True)
    ids = jax.lax.broadcasted_iota(jnp.int32, dist.shape, 1)
    idx = jnp.min(jnp.where(dist == rowmin, ids, K), axis=1, keepdims=True)
    enc = (ids == idx).astype(jnp.float32)
    q = jnp.dot(enc, cb, preferred_element_type=jnp.float32)
    diff = q - z
    m = jnp.mean(diff * diff)
    loss_ref[...] = jnp.reshape(m + COM_COST * m, (1, 1))
    qst_ref[...] = z + diff
    d1_ref[...] = jnp.tanh(jnp.dot(q, dw1_ref[...],
                                   preferred_element_type=jnp.float32)
                           + db1_ref[...])


def _dec_body(d1_ref, w2_ref, b2_ref, wd_ref, bd_ref, out_ref):
    d2 = jnp.tanh(jnp.dot(d1_ref[...], w2_ref[...],
                          preferred_element_type=jnp.float32) + b2_ref[...])
    d2b = d2.astype(jnp.bfloat16)
    acc = jnp.dot(d2b, wd_ref[0], preferred_element_type=jnp.float32)
    out_ref[...] = acc + bd_ref[0]


def kernel(inputs, enc_W, enc_b, int_enc_W1, int_enc_b1, int_enc_W2,
           int_enc_b2, codebook, int_dec_W1, int_dec_b1, int_dec_W2,
           int_dec_b2, dec_W, dec_b):
    f32 = jnp.float32

    h = pl.pallas_call(
        _enc_body,
        grid=(C,),
        in_specs=[
            pl.BlockSpec((B, G), lambda c: (0, c)),
            pl.BlockSpec((1, G, D0), lambda c: (c, 0, 0)),
            pl.BlockSpec((1, 1, D0), lambda c: (c, 0, 0)),
        ],
        out_specs=pl.BlockSpec((B, D0), lambda c: (0, c)),
        out_shape=jax.ShapeDtypeStruct((B, C * D0), f32),
    )(inputs.reshape(B, C * G), enc_W, enc_b.reshape(C, 1, D0))

    NB = 4
    z1 = pl.pallas_call(
        _z1_body,
        grid=(NB,),
        in_specs=[
            pl.BlockSpec((B, C * D0), lambda n: (0, 0)),
            pl.BlockSpec((C * D0, D1 // NB), lambda n: (0, n)),
            pl.BlockSpec((1, D1 // NB), lambda n: (0, n)),
        ],
        out_specs=pl.BlockSpec((B, D1 // NB), lambda n: (0, n)),
        out_shape=jax.ShapeDtypeStruct((B, D1), f32),
    )(h, int_enc_W1, int_enc_b1.reshape(1, D1))

    loss2d, qst, d1 = pl.pallas_call(
        _mid_body,
        in_specs=[
            pl.BlockSpec((B, D1), lambda: (0, 0)),
            pl.BlockSpec((D1, D2), lambda: (0, 0)),
            pl.BlockSpec((1, D2), lambda: (0, 0)),
            pl.BlockSpec((K, D2), lambda: (0, 0)),
            pl.BlockSpec((D2, D1), lambda: (0, 0)),
            pl.BlockSpec((1, D1), lambda: (0, 0)),
        ],
        out_specs=[
            pl.BlockSpec((1, 1), lambda: (0, 0)),
            pl.BlockSpec((B, D2), lambda: (0, 0)),
            pl.BlockSpec((B, D1), lambda: (0, 0)),
        ],
        out_shape=[
            jax.ShapeDtypeStruct((1, 1), f32),
            jax.ShapeDtypeStruct((B, D2), f32),
            jax.ShapeDtypeStruct((B, D1), f32),
        ],
    )(z1, int_enc_W2, int_enc_b2.reshape(1, D2), codebook,
      int_dec_W1, int_dec_b1.reshape(1, D1))

    d1b = d1.astype(jnp.bfloat16)
    w2b = int_dec_W2.astype(jnp.bfloat16)
    wdb = dec_W.astype(jnp.bfloat16)
    x_recon = pl.pallas_call(
        _dec_body,
        grid=(C,),
        in_specs=[
            pl.BlockSpec((B, D1), lambda c: (0, 0)),
            pl.BlockSpec((D1, D0), lambda c: (0, c)),
            pl.BlockSpec((1, D0), lambda c: (0, c)),
            pl.BlockSpec((1, D0, G), lambda c: (c, 0, 0)),
            pl.BlockSpec((1, 1, G), lambda c: (c, 0, 0)),
        ],
        out_specs=pl.BlockSpec((B, G), lambda c: (0, c)),
        out_shape=jax.ShapeDtypeStruct((B, C * G), f32),
    )(d1b, w2b, int_dec_b2.reshape(1, C * D0), wdb, dec_b.reshape(C, 1, G))

    return (loss2d[0, 0], x_recon.reshape(B, C, G), qst)


# speedup vs baseline: 1.9890x; 1.9890x over previous
import jax
import jax.numpy as jnp
from jax.experimental import pallas as pl
from jax.experimental.pallas import tpu as pltpu

B, C, G = 1024, 8, 4096
D0, D1, D2 = 512, 2048, 256
K = 1024
COM_COST = 0.25
NG = 2
GH = G // NG


def _enc_body(x_hbm, w_ref, b_ref, h_ref, xbuf, xsem):
    c = pl.program_id(0)

    @pl.when(c == 0)
    def _():
        pltpu.make_async_copy(x_hbm.at[:, 0, :], xbuf.at[0], xsem.at[0]).start()

    @pl.when(c + 1 < C)
    def _():
        slot = jax.lax.rem(c + 1, 2)
        pltpu.make_async_copy(x_hbm.at[:, c + 1, :], xbuf.at[slot],
                              xsem.at[slot]).start()

    slot = jax.lax.rem(c, 2)
    pltpu.make_async_copy(x_hbm.at[:, c, :], xbuf.at[slot],
                          xsem.at[slot]).wait()
    acc = jnp.dot(xbuf[slot], w_ref[0], preferred_element_type=jnp.float32)
    h_ref[...] = jnp.tanh(acc + b_ref[0])


def _z1_body(h_ref, w_ref, b_ref, z1_ref):
    acc = jnp.dot(h_ref[...], w_ref[...], preferred_element_type=jnp.float32)
    z1_ref[...] = jnp.tanh(acc + b_ref[...])


def _mid_body(z1_ref, w2_ref, b2_ref, cb_ref, dw1_ref, db1_ref,
              loss_ref, qst_ref, d1_ref, z_ref, zsq_ref, p_ref, dist_ref,
              idx_ref):
    preT = jax.lax.dot_general(w2_ref[...], z1_ref[...],
                               (((0,), (1,)), ((), ())),
                               preferred_element_type=jnp.float32)
    z = jnp.tanh(preT + b2_ref[...]).T
    cb = cb_ref[...]
    zz = z * z
    zsq = jnp.sum(zz.T, axis=0, keepdims=True).T
    csq = jnp.sum(cb * cb, axis=1, keepdims=True)
    p = jax.lax.dot_general(z, cb, (((1,), (1,)), ((), ())),
                            preferred_element_type=jnp.float32)
    dist = (zsq + csq.T) - 2.0 * p
    rowmin = jnp.min(dist, axis=1, keepdims=True)
    ids = jax.lax.broadcasted_iota(jnp.int32, dist.shape, 1)
    idx = jnp.min(jnp.where(dist == rowmin, ids, K), axis=1, keepdims=True)
    enc = (ids == idx).astype(jnp.float32)
    q = jnp.dot(enc, cb, preferred_element_type=jnp.float32)
    diff = q - z
    m = jnp.mean(diff * diff)
    loss_ref[...] = jnp.reshape(m + COM_COST * m, (1, 1))
    qst_ref[...] = z + diff
    d1 = jnp.tanh(jnp.dot(q, dw1_ref[...],
                          preferred_element_type=jnp.float32) + db1_ref[...])
    d1_ref[...] = d1.astype(jnp.bfloat16)
    z_ref[...] = z
    zsq_ref[...] = zsq
    p_ref[...] = p
    dist_ref[...] = dist
    idx_ref[...] = idx


def _dec_out_copy(obuf, osem, out_hbm, step, slot):
    cc = step // NG
    gg = jax.lax.rem(step, NG)
    return pltpu.make_async_copy(
        obuf.at[slot], out_hbm.at[:, cc, pl.ds(gg * GH, GH)], osem.at[slot])


def _dec_body(d1_ref, w2_ref, b2_ref, wd_ref, bd_ref, out_hbm,
              d2_scr, obuf, osem):
    c = pl.program_id(0)
    g = pl.program_id(1)
    step = c * NG + g
    slot = jax.lax.rem(step, 2)

    @pl.when(g == 0)
    def _():
        w2b = w2_ref[...].astype(jnp.bfloat16)
        d2 = jnp.tanh(jnp.dot(d1_ref[...], w2b,
                              preferred_element_type=jnp.float32) + b2_ref[...])
        d2_scr[...] = d2.astype(jnp.bfloat16)

    @pl.when(step >= 2)
    def _():
        _dec_out_copy(obuf, osem, out_hbm, step - 2, slot).wait()

    wdb = wd_ref[0].astype(jnp.bfloat16)
    acc = jnp.dot(d2_scr[...], wdb, preferred_element_type=jnp.float32)
    obuf[slot] = acc + bd_ref[0]
    _dec_out_copy(obuf, osem, out_hbm, step, slot).start()

    @pl.when(step == C * NG - 1)
    def _():
        _dec_out_copy(obuf, osem, out_hbm, step - 1, 1 - slot).wait()
        _dec_out_copy(obuf, osem, out_hbm, step, slot).wait()


def kernel(inputs, enc_W, enc_b, int_enc_W1, int_enc_b1, int_enc_W2,
           int_enc_b2, codebook, int_dec_W1, int_dec_b1, int_dec_W2,
           int_dec_b2, dec_W, dec_b):
    f32 = jnp.float32

    h = pl.pallas_call(
        _enc_body,
        grid=(C,),
        in_specs=[
            pl.BlockSpec(memory_space=pl.ANY),
            pl.BlockSpec((1, G, D0), lambda c: (c, 0, 0)),
            pl.BlockSpec((1, 1, D0), lambda c: (c, 0, 0)),
        ],
        out_specs=pl.BlockSpec((B, D0), lambda c: (0, c)),
        out_shape=jax.ShapeDtypeStruct((B, C * D0), f32),
        scratch_shapes=[
            pltpu.VMEM((2, B, G), f32),
            pltpu.SemaphoreType.DMA((2,)),
        ],
    )(inputs, enc_W, enc_b.reshape(C, 1, D0))
    h = jax.lax.optimization_barrier(h)

    NB = 4
    z1 = pl.pallas_call(
        _z1_body,
        grid=(NB,),
        in_specs=[
            pl.BlockSpec((B, C * D0), lambda n: (0, 0)),
            pl.BlockSpec((C * D0, D1 // NB), lambda n: (0, n)),
            pl.BlockSpec((1, D1 // NB), lambda n: (0, n)),
        ],
        out_specs=pl.BlockSpec((B, D1 // NB), lambda n: (0, n)),
        out_shape=jax.ShapeDtypeStruct((B, D1), f32),
    )(h, int_enc_W1, int_enc_b1.reshape(1, D1))
    z1 = jax.lax.optimization_barrier(z1)

    loss2d, qst, d1, _z, _zsq, _p, _dist, _idx = pl.pallas_call(
        _mid_body,
        in_specs=[
            pl.BlockSpec((B, D1), lambda: (0, 0)),
            pl.BlockSpec((D1, D2), lambda: (0, 0)),
            pl.BlockSpec((D2, 1), lambda: (0, 0)),
            pl.BlockSpec((K, D2), lambda: (0, 0)),
            pl.BlockSpec((D2, D1), lambda: (0, 0)),
            pl.BlockSpec((1, D1), lambda: (0, 0)),
        ],
        out_specs=[
            pl.BlockSpec((1, 1), lambda: (0, 0)),
            pl.BlockSpec((B, D2), lambda: (0, 0)),
            pl.BlockSpec((B, D1), lambda: (0, 0)),
            pl.BlockSpec((B, D2), lambda: (0, 0)),
            pl.BlockSpec((B, 1), lambda: (0, 0)),
            pl.BlockSpec((B, K), lambda: (0, 0)),
            pl.BlockSpec((B, K), lambda: (0, 0)),
            pl.BlockSpec((B, 1), lambda: (0, 0)),
        ],
        out_shape=[
            jax.ShapeDtypeStruct((1, 1), f32),
            jax.ShapeDtypeStruct((B, D2), f32),
            jax.ShapeDtypeStruct((B, D1), jnp.bfloat16),
            jax.ShapeDtypeStruct((B, D2), f32),
            jax.ShapeDtypeStruct((B, 1), f32),
            jax.ShapeDtypeStruct((B, K), f32),
            jax.ShapeDtypeStruct((B, K), f32),
            jax.ShapeDtypeStruct((B, 1), jnp.int32),
        ],
    )(z1, int_enc_W2, int_enc_b2.reshape(D2, 1), codebook,
      int_dec_W1, int_dec_b1.reshape(1, D1))

    x_recon = pl.pallas_call(
        _dec_body,
        grid=(C, NG),
        in_specs=[
            pl.BlockSpec((B, D1), lambda c, g: (0, 0)),
            pl.BlockSpec((D1, D0), lambda c, g: (0, c)),
            pl.BlockSpec((1, D0), lambda c, g: (0, c)),
            pl.BlockSpec((1, D0, GH), lambda c, g: (c, 0, g)),
            pl.BlockSpec((1, 1, GH), lambda c, g: (c, 0, g)),
        ],
        out_specs=pl.BlockSpec(memory_space=pl.ANY),
        out_shape=jax.ShapeDtypeStruct((B, C, G), f32),
        scratch_shapes=[
            pltpu.VMEM((B, D0), jnp.bfloat16),
            pltpu.VMEM((2, B, GH), f32),
            pltpu.SemaphoreType.DMA((2,)),
        ],
    )(d1, int_dec_W2, int_dec_b2.reshape(1, C * D0), dec_W,
      dec_b.reshape(C, 1, G))

    return (loss2d[0, 0], x_recon, qst)
